# baseline (device time: 86395 ns/iter reference)
import jax
import jax.numpy as jnp
from jax import lax
from jax.experimental import pallas as pl
from jax.experimental.pallas import tpu as pltpu

N_DEV = 4
R, L = 0, 1
N_STRIPE = 2


def _gelu(z):
    return 0.5 * z * (1.0 + jnp.tanh(0.7978845608 * (z + 0.044715 * z * z * z)))


def kernel(A, B):
    m, k = A.shape
    _, n = B.shape
    chunk = m // N_DEV
    half = n // 2
    w = half // N_STRIPE

    def col0(dirn, t):
        return dirn * half + t * w

    def body(a_ref, b_ref, out_ref, a_bf, b_bf, sbuf, rs_recv,
             rs_send_sems, rs_recv_sems, ag_send_sems, ag_recv_sems):
        d = lax.axis_index("i")
        right = lax.rem(d + 1, N_DEV)
        left = lax.rem(d + N_DEV - 1, N_DEV)
        nbr = (right, left)

        a_bf[...] = a_ref[...].astype(jnp.bfloat16)
        b_bf[...] = b_ref[...].astype(jnp.bfloat16)

        barrier_sem = pltpu.get_barrier_semaphore()
        for nb in (left, right):
            pl.semaphore_signal(
                barrier_sem, inc=1,
                device_id=(nb,), device_id_type=pl.DeviceIdType.MESH,
            )
        pl.semaphore_wait(barrier_sem, 2)

        def pstripe(c, dirn, t):
            a = a_bf[pl.ds(c * chunk, chunk), :]
            return jnp.dot(a, b_bf[:, pl.ds(col0(dirn, t), w)],
                           preferred_element_type=jnp.float32)

        def send_chunk(dirn, s):
            return ((d + N_DEV - 1 - s) if dirn == R else (d + 1 + s)) % N_DEV

        def recv_chunk(dirn, s):
            return ((d + 2 * N_DEV - 2 - s) if dirn == R else (d + 2 + s)) % N_DEV

        def rs_rdma(dirn, t, s):
            return pltpu.make_async_remote_copy(
                src_ref=sbuf.at[dirn, t],
                dst_ref=rs_recv.at[dirn, t, s],
                send_sem=rs_send_sems.at[dirn, t, s],
                recv_sem=rs_recv_sems.at[dirn, t, s],
                device_id=(nbr[dirn],),
                device_id_type=pl.DeviceIdType.MESH,
            )

        def ag_rdma(dirn, t, s):
            c = ((d + N_DEV - s) if dirn == R else (d + s)) % N_DEV
            sl = (pl.ds(c * chunk, chunk), pl.ds(col0(dirn, t), w))
            return pltpu.make_async_remote_copy(
                src_ref=out_ref.at[sl],
                dst_ref=out_ref.at[sl],
                send_sem=ag_send_sems.at[dirn, t, s],
                recv_sem=ag_recv_sems.at[dirn, t, s],
                device_id=(nbr[dirn],),
                device_id_type=pl.DeviceIdType.MESH,
            )

        for t in range(N_STRIPE):
            sbuf[R, t] = pstripe(send_chunk(R, 0), R, t).astype(jnp.bfloat16)
            sbuf[L, t] = pstripe(send_chunk(L, 0), L, t).astype(jnp.bfloat16)
        rs_live = {}
        for t in range(N_STRIPE):
            for dirn in (R, L):
                rs_live[dirn, t] = rs_rdma(dirn, t, 0)
                rs_live[dirn, t].start()

        ag_live = {}
        for s in range(N_DEV - 1):
            for t in range(N_STRIPE):
                ph_r = pstripe(recv_chunk(R, s), R, t)
                ph_l = pstripe(recv_chunk(L, s), L, t)
                rs_live[R, t].wait()
                rs_live[L, t].wait()
                acc_r = rs_recv[R, t, s].astype(jnp.float32) + ph_r
                acc_l = rs_recv[L, t, s].astype(jnp.float32) + ph_l
                if s < N_DEV - 2:
                    sbuf[R, t] = acc_r.astype(jnp.bfloat16)
                    sbuf[L, t] = acc_l.astype(jnp.bfloat16)
                    for dirn in (R, L):
                        rs_live[dirn, t] = rs_rdma(dirn, t, s + 1)
                        rs_live[dirn, t].start()
                else:
                    row = pl.ds(d * chunk, chunk)
                    out_ref[row, pl.ds(col0(R, t), w)] = (
                        _gelu(acc_r).astype(jnp.bfloat16))
                    out_ref[row, pl.ds(col0(L, t), w)] = (
                        _gelu(acc_l).astype(jnp.bfloat16))
                    for dirn in (R, L):
                        ag_live[dirn, t] = ag_rdma(dirn, t, 0)
                        ag_live[dirn, t].start()

        for s in range(N_DEV - 1):
            for t in range(N_STRIPE):
                ag_live[R, t].wait()
                ag_live[L, t].wait()
                if s < N_DEV - 2:
                    for dirn in (R, L):
                        ag_live[dirn, t] = ag_rdma(dirn, t, s + 1)
                        ag_live[dirn, t].start()

    return pl.pallas_call(
        body,
        out_shape=jax.ShapeDtypeStruct((m, n), jnp.bfloat16),
        in_specs=[
            pl.BlockSpec(memory_space=pltpu.VMEM),
            pl.BlockSpec(memory_space=pltpu.VMEM),
        ],
        out_specs=pl.BlockSpec(memory_space=pltpu.VMEM),
        scratch_shapes=[
            pltpu.VMEM((m, k), jnp.bfloat16),
            pltpu.VMEM((k, n), jnp.bfloat16),
            pltpu.VMEM((2, N_STRIPE, chunk, w), jnp.bfloat16),
            pltpu.VMEM((2, N_STRIPE, N_DEV - 1, chunk, w), jnp.bfloat16),
            pltpu.SemaphoreType.DMA((2, N_STRIPE, N_DEV - 1)),
            pltpu.SemaphoreType.DMA((2, N_STRIPE, N_DEV - 1)),
            pltpu.SemaphoreType.DMA((2, N_STRIPE, N_DEV - 1)),
            pltpu.SemaphoreType.DMA((2, N_STRIPE, N_DEV - 1)),
        ],
        compiler_params=pltpu.CompilerParams(collective_id=0),
    )(A, B)


# device time: 85735 ns/iter; 1.0077x vs baseline; 1.0077x over previous
import jax
import jax.numpy as jnp
from jax import lax
from jax.experimental import pallas as pl
from jax.experimental.pallas import tpu as pltpu

N_DEV = 4
R, L = 0, 1
N_STRIPE = 4


def _gelu(z):
    return 0.5 * z * (1.0 + jnp.tanh(0.7978845608 * (z + 0.044715 * z * z * z)))


def kernel(A, B):
    m, k = A.shape
    _, n = B.shape
    chunk = m // N_DEV
    half = n // 2
    w = half // N_STRIPE

    def col0(dirn, t):
        return dirn * half + t * w

    def body(a_ref, b_ref, out_ref, a_bf, b_bf, sbuf, rs_recv,
             rs_send_sems, rs_recv_sems, ag_send_sems, ag_recv_sems):
        d = lax.axis_index("i")
        right = lax.rem(d + 1, N_DEV)
        left = lax.rem(d + N_DEV - 1, N_DEV)
        nbr = (right, left)

        a_bf[...] = a_ref[...].astype(jnp.bfloat16)
        b_bf[...] = b_ref[...].astype(jnp.bfloat16)

        barrier_sem = pltpu.get_barrier_semaphore()
        for nb in (left, right):
            pl.semaphore_signal(
                barrier_sem, inc=1,
                device_id=(nb,), device_id_type=pl.DeviceIdType.MESH,
            )
        pl.semaphore_wait(barrier_sem, 2)

        def pstripe(c, dirn, t):
            a = a_bf[pl.ds(c * chunk, chunk), :]
            return jnp.dot(a, b_bf[:, pl.ds(col0(dirn, t), w)],
                           preferred_element_type=jnp.float32)

        def send_chunk(dirn, s):
            return ((d + N_DEV - 1 - s) if dirn == R else (d + 1 + s)) % N_DEV

        def recv_chunk(dirn, s):
            return ((d + 2 * N_DEV - 2 - s) if dirn == R else (d + 2 + s)) % N_DEV

        def rs_rdma(dirn, t, s):
            return pltpu.make_async_remote_copy(
                src_ref=sbuf.at[dirn, t],
                dst_ref=rs_recv.at[dirn, t, s],
                send_sem=rs_send_sems.at[dirn, t, s],
                recv_sem=rs_recv_sems.at[dirn, t, s],
                device_id=(nbr[dirn],),
                device_id_type=pl.DeviceIdType.MESH,
            )

        def ag_rdma(dirn, t, s):
            c = ((d + N_DEV - s) if dirn == R else (d + s)) % N_DEV
            sl = (pl.ds(c * chunk, chunk), pl.ds(col0(dirn, t), w))
            return pltpu.make_async_remote_copy(
                src_ref=out_ref.at[sl],
                dst_ref=out_ref.at[sl],
                send_sem=ag_send_sems.at[dirn, t, s],
                recv_sem=ag_recv_sems.at[dirn, t, s],
                device_id=(nbr[dirn],),
                device_id_type=pl.DeviceIdType.MESH,
            )

        for t in range(N_STRIPE):
            sbuf[R, t] = pstripe(send_chunk(R, 0), R, t).astype(jnp.bfloat16)
            sbuf[L, t] = pstripe(send_chunk(L, 0), L, t).astype(jnp.bfloat16)
        rs_live = {}
        for t in range(N_STRIPE):
            for dirn in (R, L):
                rs_live[dirn, t] = rs_rdma(dirn, t, 0)
                rs_live[dirn, t].start()

        ag_live = {}
        for s in range(N_DEV - 1):
            for t in range(N_STRIPE):
                ph_r = pstripe(recv_chunk(R, s), R, t)
                ph_l = pstripe(recv_chunk(L, s), L, t)
                rs_live[R, t].wait()
                rs_live[L, t].wait()
                acc_r = rs_recv[R, t, s].astype(jnp.float32) + ph_r
                acc_l = rs_recv[L, t, s].astype(jnp.float32) + ph_l
                if s < N_DEV - 2:
                    sbuf[R, t] = acc_r.astype(jnp.bfloat16)
                    sbuf[L, t] = acc_l.astype(jnp.bfloat16)
                    for dirn in (R, L):
                        rs_live[dirn, t] = rs_rdma(dirn, t, s + 1)
                        rs_live[dirn, t].start()
                else:
                    row = pl.ds(d * chunk, chunk)
                    out_ref[row, pl.ds(col0(R, t), w)] = (
                        _gelu(acc_r).astype(jnp.bfloat16))
                    out_ref[row, pl.ds(col0(L, t), w)] = (
                        _gelu(acc_l).astype(jnp.bfloat16))
                    for dirn in (R, L):
                        ag_live[dirn, t] = ag_rdma(dirn, t, 0)
                        ag_live[dirn, t].start()

        for s in range(N_DEV - 1):
            for t in range(N_STRIPE):
                ag_live[R, t].wait()
                ag_live[L, t].wait()
                if s < N_DEV - 2:
                    for dirn in (R, L):
                        ag_live[dirn, t] = ag_rdma(dirn, t, s + 1)
                        ag_live[dirn, t].start()

    return pl.pallas_call(
        body,
        out_shape=jax.ShapeDtypeStruct((m, n), jnp.bfloat16),
        in_specs=[
            pl.BlockSpec(memory_space=pltpu.VMEM),
            pl.BlockSpec(memory_space=pltpu.VMEM),
        ],
        out_specs=pl.BlockSpec(memory_space=pltpu.VMEM),
        scratch_shapes=[
            pltpu.VMEM((m, k), jnp.bfloat16),
            pltpu.VMEM((k, n), jnp.bfloat16),
            pltpu.VMEM((2, N_STRIPE, chunk, w), jnp.bfloat16),
            pltpu.VMEM((2, N_STRIPE, N_DEV - 1, chunk, w), jnp.bfloat16),
            pltpu.SemaphoreType.DMA((2, N_STRIPE, N_DEV - 1)),
            pltpu.SemaphoreType.DMA((2, N_STRIPE, N_DEV - 1)),
            pltpu.SemaphoreType.DMA((2, N_STRIPE, N_DEV - 1)),
            pltpu.SemaphoreType.DMA((2, N_STRIPE, N_DEV - 1)),
        ],
        compiler_params=pltpu.CompilerParams(collective_id=0),
    )(A, B)


# device time: 85103 ns/iter; 1.0152x vs baseline; 1.0074x over previous
import jax
import jax.numpy as jnp
from jax import lax
from jax.experimental import pallas as pl
from jax.experimental.pallas import tpu as pltpu

N_DEV = 4
R, L = 0, 1
N_STRIPE = 4


def _gelu(z):
    return 0.5 * z * (1.0 + jnp.tanh(0.7978845608 * (z + 0.044715 * z * z * z)))


def kernel(A, B):
    m, k = A.shape
    _, n = B.shape
    chunk = m // N_DEV
    half = n // 2
    w = half // N_STRIPE

    def col0(dirn, t):
        return dirn * half + t * w

    def body(a_ref, b_ref, out_ref, a_bf, b_bf, sbuf, rs_recv,
             rs_send_sems, rs_recv_sems, ag_send_sems, ag_recv_sems):
        d = lax.axis_index("i")
        right = lax.rem(d + 1, N_DEV)
        left = lax.rem(d + N_DEV - 1, N_DEV)
        nbr = (right, left)

        barrier_sem = pltpu.get_barrier_semaphore()
        for nb in (left, right):
            pl.semaphore_signal(
                barrier_sem, inc=1,
                device_id=(nb,), device_id_type=pl.DeviceIdType.MESH,
            )
        pl.semaphore_wait(barrier_sem, 2)

        def cast_a_chunk(c):
            sl = pl.ds(c * chunk, chunk)
            a_bf[sl, :] = a_ref[sl, :].astype(jnp.bfloat16)

        b_bf[...] = b_ref[...].astype(jnp.bfloat16)
        cast_a_chunk(lax.rem(d + N_DEV - 1, N_DEV))
        cast_a_chunk(lax.rem(d + 1, N_DEV))

        def pstripe(c, dirn, t):
            a = a_bf[pl.ds(c * chunk, chunk), :]
            return jnp.dot(a, b_bf[:, pl.ds(col0(dirn, t), w)],
                           preferred_element_type=jnp.float32)

        def send_chunk(dirn, s):
            return ((d + N_DEV - 1 - s) if dirn == R else (d + 1 + s)) % N_DEV

        def recv_chunk(dirn, s):
            return ((d + 2 * N_DEV - 2 - s) if dirn == R else (d + 2 + s)) % N_DEV

        def rs_rdma(dirn, t, s):
            return pltpu.make_async_remote_copy(
                src_ref=sbuf.at[dirn, t],
                dst_ref=rs_recv.at[dirn, t, s],
                send_sem=rs_send_sems.at[dirn, t, s],
                recv_sem=rs_recv_sems.at[dirn, t, s],
                device_id=(nbr[dirn],),
                device_id_type=pl.DeviceIdType.MESH,
            )

        def ag_rdma(dirn, t, s):
            c = ((d + N_DEV - s) if dirn == R else (d + s)) % N_DEV
            sl = (pl.ds(c * chunk, chunk), pl.ds(col0(dirn, t), w))
            return pltpu.make_async_remote_copy(
                src_ref=out_ref.at[sl],
                dst_ref=out_ref.at[sl],
                send_sem=ag_send_sems.at[dirn, t, s],
                recv_sem=ag_recv_sems.at[dirn, t, s],
                device_id=(nbr[dirn],),
                device_id_type=pl.DeviceIdType.MESH,
            )

        rs_live = {}
        for t in range(N_STRIPE):
            for dirn in (R, L):
                sbuf[dirn, t] = pstripe(send_chunk(dirn, 0), dirn, t).astype(
                    jnp.bfloat16)
                rs_live[dirn, t] = rs_rdma(dirn, t, 0)
                rs_live[dirn, t].start()

        cast_a_chunk(d)
        cast_a_chunk(lax.rem(d + 2, N_DEV))

        ag_live = {}
        for s in range(N_DEV - 1):
            for t in range(N_STRIPE):
                ph_r = pstripe(recv_chunk(R, s), R, t)
                ph_l = pstripe(recv_chunk(L, s), L, t)
                rs_live[R, t].wait()
                rs_live[L, t].wait()
                acc_r = rs_recv[R, t, s].astype(jnp.float32) + ph_r
                acc_l = rs_recv[L, t, s].astype(jnp.float32) + ph_l
                if s < N_DEV - 2:
                    sbuf[R, t] = acc_r.astype(jnp.bfloat16)
                    sbuf[L, t] = acc_l.astype(jnp.bfloat16)
                    for dirn in (R, L):
                        rs_live[dirn, t] = rs_rdma(dirn, t, s + 1)
                        rs_live[dirn, t].start()
                else:
                    row = pl.ds(d * chunk, chunk)
                    out_ref[row, pl.ds(col0(R, t), w)] = (
                        _gelu(acc_r).astype(jnp.bfloat16))
                    out_ref[row, pl.ds(col0(L, t), w)] = (
                        _gelu(acc_l).astype(jnp.bfloat16))
                    for dirn in (R, L):
                        ag_live[dirn, t] = ag_rdma(dirn, t, 0)
                        ag_live[dirn, t].start()

        for s in range(N_DEV - 1):
            for t in range(N_STRIPE):
                ag_live[R, t].wait()
                ag_live[L, t].wait()
                if s < N_DEV - 2:
                    for dirn in (R, L):
                        ag_live[dirn, t] = ag_rdma(dirn, t, s + 1)
                        ag_live[dirn, t].start()

    return pl.pallas_call(
        body,
        out_shape=jax.ShapeDtypeStruct((m, n), jnp.bfloat16),
        in_specs=[
            pl.BlockSpec(memory_space=pltpu.VMEM),
            pl.BlockSpec(memory_space=pltpu.VMEM),
        ],
        out_specs=pl.BlockSpec(memory_space=pltpu.VMEM),
        scratch_shapes=[
            pltpu.VMEM((m, k), jnp.bfloat16),
            pltpu.VMEM((k, n), jnp.bfloat16),
            pltpu.VMEM((2, N_STRIPE, chunk, w), jnp.bfloat16),
            pltpu.VMEM((2, N_STRIPE, N_DEV - 1, chunk, w), jnp.bfloat16),
            pltpu.SemaphoreType.DMA((2, N_STRIPE, N_DEV - 1)),
            pltpu.SemaphoreType.DMA((2, N_STRIPE, N_DEV - 1)),
            pltpu.SemaphoreType.DMA((2, N_STRIPE, N_DEV - 1)),
            pltpu.SemaphoreType.DMA((2, N_STRIPE, N_DEV - 1)),
        ],
        compiler_params=pltpu.CompilerParams(collective_id=0),
    )(A, B)


# device time: 84769 ns/iter; 1.0192x vs baseline; 1.0039x over previous
import jax
import jax.numpy as jnp
from jax import lax
from jax.experimental import pallas as pl
from jax.experimental.pallas import tpu as pltpu

N_DEV = 4
R, L = 0, 1
N_STRIPE = 4


def _gelu(z):
    return 0.5 * z * (1.0 + jnp.tanh(0.7978845608 * (z + 0.044715 * z * z * z)))


def kernel(A, B):
    m, k = A.shape
    _, n = B.shape
    chunk = m // N_DEV
    half = n // 2
    w = half // N_STRIPE

    def col0(dirn, t):
        return dirn * half + t * w

    def body(a_ref, b_ref, out_ref, a_bf, b_bf, sbuf, rs_recv,
             rs_send_sems, rs_recv_sems, ag_send_sems, ag_recv_sems):
        d = lax.axis_index("i")
        right = lax.rem(d + 1, N_DEV)
        left = lax.rem(d + N_DEV - 1, N_DEV)
        nbr = (right, left)

        barrier_sem = pltpu.get_barrier_semaphore()
        for nb in (left, right):
            pl.semaphore_signal(
                barrier_sem, inc=1,
                device_id=(nb,), device_id_type=pl.DeviceIdType.MESH,
            )
        pl.semaphore_wait(barrier_sem, 2)

        def cast_a_chunk(c):
            sl = pl.ds(c * chunk, chunk)
            a_bf[sl, :] = a_ref[sl, :].astype(jnp.bfloat16)

        def cast_b_stripe(dirn, t):
            sl = pl.ds(col0(dirn, t), w)
            b_bf[:, sl] = b_ref[:, sl].astype(jnp.bfloat16)

        def pstripe(c, dirn, t):
            a = a_bf[pl.ds(c * chunk, chunk), :]
            return jnp.dot(a, b_bf[:, pl.ds(col0(dirn, t), w)],
                           preferred_element_type=jnp.float32)

        def send_chunk(dirn, s):
            return ((d + N_DEV - 1 - s) if dirn == R else (d + 1 + s)) % N_DEV

        def recv_chunk(dirn, s):
            return ((d + 2 * N_DEV - 2 - s) if dirn == R else (d + 2 + s)) % N_DEV

        def rs_rdma(dirn, t, s):
            return pltpu.make_async_remote_copy(
                src_ref=sbuf.at[dirn, t],
                dst_ref=rs_recv.at[dirn, t, s],
                send_sem=rs_send_sems.at[dirn, t, s],
                recv_sem=rs_recv_sems.at[dirn, t, s],
                device_id=(nbr[dirn],),
                device_id_type=pl.DeviceIdType.MESH,
            )

        def ag_rdma(dirn, t, s):
            c = ((d + N_DEV - s) if dirn == R else (d + s)) % N_DEV
            sl = (pl.ds(c * chunk, chunk), pl.ds(col0(dirn, t), w))
            return pltpu.make_async_remote_copy(
                src_ref=out_ref.at[sl],
                dst_ref=out_ref.at[sl],
                send_sem=ag_send_sems.at[dirn, t, s],
                recv_sem=ag_recv_sems.at[dirn, t, s],
                device_id=(nbr[dirn],),
                device_id_type=pl.DeviceIdType.MESH,
            )

        rs_live = {}
        for t in range(N_STRIPE):
            for dirn in (R, L):
                if t == 0:
                    cast_a_chunk(send_chunk(dirn, 0))
                cast_b_stripe(dirn, t)
                sbuf[dirn, t] = pstripe(send_chunk(dirn, 0), dirn, t).astype(
                    jnp.bfloat16)
                rs_live[dirn, t] = rs_rdma(dirn, t, 0)
                rs_live[dirn, t].start()

        cast_a_chunk(d)
        cast_a_chunk(lax.rem(d + 2, N_DEV))

        ag_live = {}
        for s in range(N_DEV - 1):
            for t in range(N_STRIPE):
                ph_r = pstripe(recv_chunk(R, s), R, t)
                ph_l = pstripe(recv_chunk(L, s), L, t)
                rs_live[R, t].wait()
                rs_live[L, t].wait()
                acc_r = rs_recv[R, t, s].astype(jnp.float32) + ph_r
                acc_l = rs_recv[L, t, s].astype(jnp.float32) + ph_l
                if s < N_DEV - 2:
                    sbuf[R, t] = acc_r.astype(jnp.bfloat16)
                    sbuf[L, t] = acc_l.astype(jnp.bfloat16)
                    for dirn in (R, L):
                        rs_live[dirn, t] = rs_rdma(dirn, t, s + 1)
                        rs_live[dirn, t].start()
                else:
                    row = pl.ds(d * chunk, chunk)
                    out_ref[row, pl.ds(col0(R, t), w)] = (
                        _gelu(acc_r).astype(jnp.bfloat16))
                    out_ref[row, pl.ds(col0(L, t), w)] = (
                        _gelu(acc_l).astype(jnp.bfloat16))
                    for dirn in (R, L):
                        ag_live[dirn, t] = ag_rdma(dirn, t, 0)
                        ag_live[dirn, t].start()

        for s in range(N_DEV - 1):
            for t in range(N_STRIPE):
                ag_live[R, t].wait()
                ag_live[L, t].wait()
                if s < N_DEV - 2:
                    for dirn in (R, L):
                        ag_live[dirn, t] = ag_rdma(dirn, t, s + 1)
                        ag_live[dirn, t].start()

    return pl.pallas_call(
        body,
        out_shape=jax.ShapeDtypeStruct((m, n), jnp.bfloat16),
        in_specs=[
            pl.BlockSpec(memory_space=pltpu.VMEM),
            pl.BlockSpec(memory_space=pltpu.VMEM),
        ],
        out_specs=pl.BlockSpec(memory_space=pltpu.VMEM),
        scratch_shapes=[
            pltpu.VMEM((m, k), jnp.bfloat16),
            pltpu.VMEM((k, n), jnp.bfloat16),
            pltpu.VMEM((2, N_STRIPE, chunk, w), jnp.bfloat16),
            pltpu.VMEM((2, N_STRIPE, N_DEV - 1, chunk, w), jnp.bfloat16),
            pltpu.SemaphoreType.DMA((2, N_STRIPE, N_DEV - 1)),
            pltpu.SemaphoreType.DMA((2, N_STRIPE, N_DEV - 1)),
            pltpu.SemaphoreType.DMA((2, N_STRIPE, N_DEV - 1)),
            pltpu.SemaphoreType.DMA((2, N_STRIPE, N_DEV - 1)),
        ],
        compiler_params=pltpu.CompilerParams(collective_id=0),
    )(A, B)


# device time: 79387 ns/iter; 1.0883x vs baseline; 1.0678x over previous
import jax
import jax.numpy as jnp
from jax import lax
from jax.experimental import pallas as pl
from jax.experimental.pallas import tpu as pltpu

N_DEV = 4
R, L = 0, 1
N_STRIPE = 4


def _gelu(z):
    return 0.5 * z * (1.0 + jnp.tanh(0.7978845608 * (z + 0.044715 * z * z * z)))


def kernel(A, B):
    m, k = A.shape
    _, n = B.shape
    chunk = m // N_DEV
    half = n // 2
    w = half // N_STRIPE

    def col0(dirn, t):
        return dirn * half + t * w

    def bslot(dirn, t):
        return 4 + t * 2 + dirn

    def body(a_ref, b_ref, out_ref, a_f32, b_f32, a_bf, b_bf, own, sbuf,
             rs_recv, ldma_sems, rs_send_sems, rs_recv_sems, ag_send_sems,
             ag_recv_sems):
        d = lax.axis_index("i")
        right = lax.rem(d + 1, N_DEV)
        left = lax.rem(d + N_DEV - 1, N_DEV)
        nbr = (right, left)

        a_order = (lax.rem(d + N_DEV - 1, N_DEV), lax.rem(d + 1, N_DEV),
                   d, lax.rem(d + 2, N_DEV))

        def a_copy(o):
            sl = pl.ds(a_order[o] * chunk, chunk)
            return pltpu.make_async_copy(a_ref.at[sl, :], a_f32.at[o % 2],
                                         ldma_sems.at[o % 2])

        def b_copy(u):
            dirn, t = u % 2, u // 2
            sl = pl.ds(col0(dirn, t), w)
            return pltpu.make_async_copy(b_ref.at[:, sl], b_f32.at[u % 2],
                                         ldma_sems.at[2 + u % 2])

        a_copy(0).start()
        a_copy(1).start()
        b_copy(0).start()
        b_copy(1).start()

        barrier_sem = pltpu.get_barrier_semaphore()
        for nb in (left, right):
            pl.semaphore_signal(
                barrier_sem, inc=1,
                device_id=(nb,), device_id_type=pl.DeviceIdType.MESH,
            )
        pl.semaphore_wait(barrier_sem, 2)

        def wait_a_chunk(o):
            a_copy(o).wait()
            sl = pl.ds(a_order[o] * chunk, chunk)
            a_bf[sl, :] = a_f32[o % 2].astype(jnp.bfloat16)
            if o + 2 < 4:
                a_copy(o + 2).start()

        def wait_b_stripe(u):
            b_copy(u).wait()
            dirn, t = u % 2, u // 2
            b_bf[:, pl.ds(col0(dirn, t), w)] = b_f32[u % 2].astype(
                jnp.bfloat16)
            if u + 2 < 2 * N_STRIPE:
                b_copy(u + 2).start()

        def pstripe(c, dirn, t):
            a = a_bf[pl.ds(c * chunk, chunk), :]
            return jnp.dot(a, b_bf[:, pl.ds(col0(dirn, t), w)],
                           preferred_element_type=jnp.float32)

        def send_chunk(dirn, s):
            return ((d + N_DEV - 1 - s) if dirn == R else (d + 1 + s)) % N_DEV

        def recv_chunk(dirn, s):
            return ((d + 2 * N_DEV - 2 - s) if dirn == R else (d + 2 + s)) % N_DEV

        def rs_rdma(dirn, t, s):
            return pltpu.make_async_remote_copy(
                src_ref=sbuf.at[dirn, t],
                dst_ref=rs_recv.at[dirn, t, s],
                send_sem=rs_send_sems.at[dirn, t, s],
                recv_sem=rs_recv_sems.at[dirn, t, s],
                device_id=(nbr[dirn],),
                device_id_type=pl.DeviceIdType.MESH,
            )

        def ag_rdma(src, c, dirn, t, sem_idx, target):
            sl = (pl.ds(c * chunk, chunk), pl.ds(col0(dirn, t), w))
            return pltpu.make_async_remote_copy(
                src_ref=src,
                dst_ref=out_ref.at[sl],
                send_sem=ag_send_sems.at[dirn, t, sem_idx],
                recv_sem=ag_recv_sems.at[dirn, t, sem_idx],
                device_id=(target,),
                device_id_type=pl.DeviceIdType.MESH,
            )

        rs_live = {}
        for t in range(N_STRIPE):
            for dirn in (R, L):
                if t == 0:
                    wait_a_chunk(dirn)
                wait_b_stripe(t * 2 + dirn)
                sbuf[dirn, t] = pstripe(send_chunk(dirn, 0), dirn, t).astype(
                    jnp.bfloat16)
                rs_live[dirn, t] = rs_rdma(dirn, t, 0)
                rs_live[dirn, t].start()

        wait_a_chunk(2)
        wait_a_chunk(3)

        ag_live = {}
        for s in range(N_DEV - 1):
            for t in range(N_STRIPE):
                ph_r = pstripe(recv_chunk(R, s), R, t)
                ph_l = pstripe(recv_chunk(L, s), L, t)
                rs_live[R, t].wait()
                acc_r = rs_recv[R, t, s].astype(jnp.float32) + ph_r
                if s < N_DEV - 2:
                    sbuf[R, t] = acc_r.astype(jnp.bfloat16)
                    rs_live[R, t] = rs_rdma(R, t, s + 1)
                    rs_live[R, t].start()
                rs_live[L, t].wait()
                acc_l = rs_recv[L, t, s].astype(jnp.float32) + ph_l
                if s < N_DEV - 2:
                    sbuf[L, t] = acc_l.astype(jnp.bfloat16)
                    rs_live[L, t] = rs_rdma(L, t, s + 1)
                    rs_live[L, t].start()
                else:
                    csl = pl.ds(col0(R, t), w)
                    own[:, csl] = _gelu(acc_r).astype(jnp.bfloat16)
                    ag_live[R, t, 0] = ag_rdma(own.at[:, csl], d, R, t, 0,
                                               right)
                    ag_live[R, t, 0].start()
                    ag_live[R, t, 2] = ag_rdma(own.at[:, csl], d, R, t, 2,
                                               left)
                    ag_live[R, t, 2].start()
                    csl = pl.ds(col0(L, t), w)
                    own[:, csl] = _gelu(acc_l).astype(jnp.bfloat16)
                    ag_live[L, t, 0] = ag_rdma(own.at[:, csl], d, L, t, 0,
                                               left)
                    ag_live[L, t, 0].start()
                    ag_live[L, t, 2] = ag_rdma(own.at[:, csl], d, L, t, 2,
                                               right)
                    ag_live[L, t, 2].start()

        own_out = pltpu.make_async_copy(
            own, out_ref.at[pl.ds(d * chunk, chunk), :], ldma_sems.at[4])
        own_out.start()

        cm1 = lax.rem(d + N_DEV - 1, N_DEV)
        cp1 = lax.rem(d + 1, N_DEV)
        for t in range(N_STRIPE):
            ag_live[R, t, 0].wait()
            src = out_ref.at[pl.ds(cm1 * chunk, chunk), pl.ds(col0(R, t), w)]
            ag_live[R, t, 1] = ag_rdma(src, cm1, R, t, 1, right)
            ag_live[R, t, 1].start()
            ag_live[L, t, 0].wait()
            src = out_ref.at[pl.ds(cp1 * chunk, chunk), pl.ds(col0(L, t), w)]
            ag_live[L, t, 1] = ag_rdma(src, cp1, L, t, 1, left)
            ag_live[L, t, 1].start()
        for t in range(N_STRIPE):
            ag_live[R, t, 2].wait()
            ag_live[L, t, 2].wait()
        for t in range(N_STRIPE):
            ag_live[R, t, 1].wait()
            ag_live[L, t, 1].wait()
        own_out.wait()

    return pl.pallas_call(
        body,
        out_shape=jax.ShapeDtypeStruct((m, n), jnp.bfloat16),
        in_specs=[
            pl.BlockSpec(memory_space=pl.ANY),
            pl.BlockSpec(memory_space=pl.ANY),
        ],
        out_specs=pl.BlockSpec(memory_space=pltpu.MemorySpace.HBM),
        scratch_shapes=[
            pltpu.VMEM((2, chunk, k), jnp.float32),
            pltpu.VMEM((2, k, w), jnp.float32),
            pltpu.VMEM((m, k), jnp.bfloat16),
            pltpu.VMEM((k, n), jnp.bfloat16),
            pltpu.VMEM((chunk, n), jnp.bfloat16),
            pltpu.VMEM((2, N_STRIPE, chunk, w), jnp.bfloat16),
            pltpu.VMEM((2, N_STRIPE, N_DEV - 1, chunk, w), jnp.bfloat16),
            pltpu.SemaphoreType.DMA((5,)),
            pltpu.SemaphoreType.DMA((2, N_STRIPE, N_DEV - 1)),
            pltpu.SemaphoreType.DMA((2, N_STRIPE, N_DEV - 1)),
            pltpu.SemaphoreType.DMA((2, N_STRIPE, N_DEV - 1)),
            pltpu.SemaphoreType.DMA((2, N_STRIPE, N_DEV - 1)),
        ],
        compiler_params=pltpu.CompilerParams(collective_id=0),
    )(
        pltpu.with_memory_space_constraint(A, pltpu.MemorySpace.HBM),
        pltpu.with_memory_space_constraint(B, pltpu.MemorySpace.HBM),
    )
